# baseline (speedup 1.0000x reference)
import functools

import jax
import jax.numpy as jnp
from jax import lax
from jax.experimental import pallas as pl
from jax.experimental.pallas import tpu as pltpu
from jax.experimental.pallas import tpu_sc as plsc

NUM_CLASSES = 1000000
EMBED_DIM = 16
BATCH = 16384

_INFO = plsc.get_sparse_core_info()
_NC, _NS = _INFO.num_cores, _INFO.num_subcores
_NW = _NC * _NS
_NWIN = (NUM_CLASSES + 127) // 128
_WCHUNK = 16
_CCOLS = _WCHUNK * 128
_NCHUNK = 16
_LOCAL_CAP = 1024
_RES_CAP = 128
_OUT_ROWS = BATCH + 16

_mesh = plsc.VectorSubcoreMesh(core_axis_name="c", subcore_axis_name="s")


@functools.partial(
    pl.kernel,
    mesh=_mesh,
    compiler_params=pltpu.CompilerParams(
        needs_layout_passes=False,
        disable_bounds_checks=True,
        disable_semaphore_checks=True,
        skip_device_barrier=True,
    ),
    out_type=jax.ShapeDtypeStruct((_OUT_ROWS, 128), jnp.float32),
    scratch_types=[
        pltpu.VMEM((128, 128), jnp.int32),
        pltpu.VMEM((_LOCAL_CAP,), jnp.int32),
        pltpu.VMEM((_LOCAL_CAP,), jnp.int32),
        pltpu.VMEM((2, EMBED_DIM, _CCOLS), jnp.float32),
        pltpu.VMEM((256,), jnp.int32),
        pltpu.VMEM((256,), jnp.int32),
        pltpu.VMEM((_RES_CAP, 128), jnp.float32),
        pltpu.VMEM((_RES_CAP,), jnp.int32),
        pltpu.SemaphoreType.DMA,
        pltpu.SemaphoreType.DMA,
        pltpu.SemaphoreType.DMA,
    ],
)
def _gather_kernel(labels_hbm, tablet_hbm, out_hbm, lbl_all, loc_lbl, loc_pos,
                   chunks, ch_lbl, ch_pos, res_v, pos_v, sem0, sem1, scat_sem):
    wid = lax.axis_index("s") * _NC + lax.axis_index("c")
    lanes = lax.iota(jnp.int32, 16)
    csems = (sem0, sem1)

    start_w = (wid * _NWIN) >> 5
    end_w = ((wid + 1) * _NWIN) >> 5
    lo_cls = start_w << 7
    hi_cls = end_w << 7

    def _chunk_base(k):
        return jnp.minimum(start_w + k * _WCHUNK, end_w - _WCHUNK)

    def _enqueue_chunk(k, p):
        cb = pl.multiple_of(_chunk_base(k) << 7, 128)
        pltpu.async_copy(
            tablet_hbm.at[pl.ds(0, EMBED_DIM), pl.ds(cb, _CCOLS)],
            chunks.at[p],
            csems[p],
        )

    pltpu.sync_copy(labels_hbm, lbl_all)
    _enqueue_chunk(0, 0)
    _enqueue_chunk(1, 1)

    @pl.loop(0, 128, init_carry=jnp.int32(0))
    def _scan(r, off):
        for c in range(8):
            l16 = lbl_all[r, pl.ds(c * 16, 16)]
            p16 = r * 128 + c * 16 + lanes
            m = (l16 >= lo_cls) & (l16 < hi_cls)
            plsc.store_compressed(loc_lbl.at[pl.ds(off, 16)], l16, mask=m)
            plsc.store_compressed(loc_pos.at[pl.ds(off, 16)], p16, mask=m)
            cnt = plsc.all_reduce_population_count(m)[0]
            off = jnp.minimum(off + cnt, _LOCAL_CAP - 16)
        return off

    nloc = _scan
    ngrp_loc = (nloc + 15) >> 4

    @pl.loop(0, _NCHUNK // 2)
    def _stream(k2):
        for p in range(2):
            k = k2 * 2 + p
            cb_cls = _chunk_base(k) << 7

            @pl.loop(0, ngrp_loc, init_carry=jnp.int32(0))
            def _compress(j, r):
                ll = loc_lbl[pl.ds(j * 16, 16)]
                pp = loc_pos[pl.ds(j * 16, 16)]
                m = (
                    (ll >= cb_cls)
                    & (ll < cb_cls + _CCOLS)
                    & (j * 16 + lanes < nloc)
                )
                plsc.store_compressed(ch_lbl.at[pl.ds(r, 16)], ll, mask=m)
                plsc.store_compressed(ch_pos.at[pl.ds(r, 16)], pp, mask=m)
                cnt = plsc.all_reduce_population_count(m)[0]
                return jnp.minimum(r + cnt, 224)

            nres = _compress

            @pl.when(k > 0)
            def _():
                pltpu.make_async_copy(
                    out_hbm.at[pl.ds(0, _RES_CAP)], res_v, scat_sem
                ).wait()

            for q in range(_RES_CAP // 16):
                pos_v[pl.ds(q * 16, 16)] = BATCH + lanes

            pltpu.make_async_copy(
                tablet_hbm.at[pl.ds(0, EMBED_DIM), pl.ds(0, _CCOLS)],
                chunks.at[p],
                csems[p],
            ).wait()

            ngrp_res = jnp.minimum((nres + 15) >> 4, _RES_CAP // 16)

            @pl.loop(0, ngrp_res)
            def _extract(g):
                cl = ch_lbl[pl.ds(g * 16, 16)]
                cp = ch_pos[pl.ds(g * 16, 16)]
                valid = g * 16 + lanes < nres
                coff = jnp.clip(cl - cb_cls, 0, _CCOLS - 1)
                pos_v[pl.ds(g * 16, 16)] = jnp.where(valid, cp, BATCH + lanes)
                for b in range(16):
                    cf = jnp.full((16,), coff[b], jnp.int32)
                    vals = plsc.load_gather(chunks.at[p], [lanes, cf])
                    res_v[g * 16 + b, pl.ds(0, EMBED_DIM)] = vals

            pltpu.async_copy(res_v, out_hbm.at[pos_v], scat_sem)

            @pl.when(k + 2 < _NCHUNK)
            def _():
                _enqueue_chunk(k + 2, p)

    pltpu.make_async_copy(
        out_hbm.at[pl.ds(0, _RES_CAP)], res_v, scat_sem
    ).wait()


def kernel(labels, table):
    labels2d = labels.astype(jnp.int32).reshape(128, 128)
    padded = _gather_kernel(labels2d, table.T)
    return padded[:BATCH, :EMBED_DIM]


# speedup vs baseline: 3.3492x; 3.3492x over previous
import functools

import jax
import jax.numpy as jnp
from jax import lax
from jax.experimental import pallas as pl
from jax.experimental.pallas import tpu as pltpu
from jax.experimental.pallas import tpu_sc as plsc

NUM_CLASSES = 1000000
EMBED_DIM = 16
BATCH = 16384

_INFO = plsc.get_sparse_core_info()
_NC, _NS = _INFO.num_cores, _INFO.num_subcores
_NW = _NC * _NS
_B_PER_W = BATCH // _NW
_G = 16
_P = 3
_NGRP = _B_PER_W // _G

_mesh = plsc.VectorSubcoreMesh(core_axis_name="c", subcore_axis_name="s")


@functools.partial(
    pl.kernel,
    mesh=_mesh,
    compiler_params=pltpu.CompilerParams(
        needs_layout_passes=False,
        disable_bounds_checks=True,
        disable_semaphore_checks=True,
        skip_device_barrier=True,
    ),
    out_type=jax.ShapeDtypeStruct((EMBED_DIM, BATCH), jnp.float32),
    scratch_types=[
        pltpu.VMEM((8, 128), jnp.int32),
        pltpu.VMEM((_P, _G, EMBED_DIM, 128), jnp.float32),
        pltpu.VMEM((EMBED_DIM, _B_PER_W), jnp.float32),
    ]
    + [pltpu.SemaphoreType.DMA] * _P,
)
def _gather_kernel(labels_hbm, tablet_hbm, outt_hbm, lbl_v, slots, out_v,
                   *sems):
    wid = lax.axis_index("s") * _NC + lax.axis_index("c")
    base = wid * _B_PER_W
    pltpu.sync_copy(labels_hbm.at[pl.ds((wid // 2) * 8, 8), pl.ds(0, 128)], lbl_v)
    row0 = (wid % 2) * 4
    lanes = lax.iota(jnp.int32, 16)
    per_row = 128 // _G

    def _group_labels(g):
        return lbl_v[row0 + g // per_row, pl.ds((g % per_row) * _G, _G)]

    def _enqueue_group(g, p, sem):
        lblg = _group_labels(g)
        cbg = (lblg >> 7) << 7
        for b in range(_G):
            cbase = pl.multiple_of(cbg[b], 128)
            pltpu.async_copy(
                tablet_hbm.at[pl.ds(0, EMBED_DIM), pl.ds(cbase, 128)],
                slots.at[p, b],
                sem,
            )

    def _wait_extract_group(g, p, sem):
        lblg = _group_labels(g)
        coffg = lblg & 127
        for b in range(_G):
            pltpu.make_async_copy(
                tablet_hbm.at[pl.ds(0, EMBED_DIM), pl.ds(0, 128)],
                slots.at[p, b],
                sem,
            ).wait()
            coff = jnp.full((16,), coffg[b], jnp.int32)
            vals = plsc.load_gather(slots.at[p, b], [lanes, coff])
            plsc.store_scatter(
                out_v, [lanes, jnp.full((16,), g * _G + b, jnp.int32)], vals
            )

    for p in range(_P):
        _enqueue_group(p, p, sems[p])

    @pl.loop(0, _NGRP // _P)
    def _body(gp):
        for p in range(_P):
            g = gp * _P + p
            _wait_extract_group(g, p, sems[p])

            @pl.when(g + _P < _NGRP)
            def _():
                _enqueue_group(g + _P, p, sems[p])

    for g in range((_NGRP // _P) * _P, _NGRP):
        p = g % _P
        _wait_extract_group(g, p, sems[p])

    pltpu.sync_copy(
        out_v, outt_hbm.at[pl.ds(0, EMBED_DIM), pl.ds(base, _B_PER_W)]
    )


def kernel(labels, table):
    labels2d = labels.astype(jnp.int32).reshape(BATCH // 128, 128)
    outt = _gather_kernel(labels2d, table.T)
    return outt.T
